# baseline (speedup 1.0000x reference)
import jax
import jax.numpy as jnp
from jax.experimental import pallas as pl
from jax.experimental.pallas import tpu as pltpu

NP = 10240
RB = 256
CB = 256
KNN = 20
BW = 32
NSEG = 16
BIG = float(3e38)
MASKED = float(1e30)


def _knn_merge(bv, bi, dch, ich):
    cv = jnp.concatenate([bv, dch], axis=1)
    ci = jnp.concatenate([bi, ich], axis=1)
    ms, cs = [], []
    for _ in range(KNN):
        m = jnp.min(cv, axis=1, keepdims=True)
        tie = jnp.where(cv == m, ci, BIG)
        c = jnp.min(tie, axis=1, keepdims=True)
        sel = ci == c
        cv = jnp.where(sel, BIG, cv)
        ms.append(m)
        cs.append(c)
    rb = bv.shape[0]
    ms.append(jnp.full((rb, BW - KNN), BIG, jnp.float32))
    cs.append(jnp.full((rb, BW - KNN), BIG, jnp.float32))
    return jnp.concatenate(ms, axis=1), jnp.concatenate(cs, axis=1)


def _dot(a, b):
    return jax.lax.dot_general(a, b, (((1,), (0,)), ((), ())),
                               preferred_element_type=jnp.float32)


def _limbs(x):
    hi = x.astype(jnp.bfloat16)
    r1 = x - hi.astype(jnp.float32)
    lo = r1.astype(jnp.bfloat16)
    r2 = r1 - lo.astype(jnp.float32)
    lo2 = r2.astype(jnp.bfloat16)
    return hi, lo, lo2


def _make_edgeconv_body(d_in, d_out, n_layers, split_mlp=False):
    def body(win_ref, xall_ref, xt_ref, sqc_all_ref, bcol_ref,
             xblk_ref, sqr_ref, brow_ref, *rest):
        wrefs = rest[:-1]
        out_ref = rest[-1]
        blk = pl.program_id(0)
        w0 = win_ref[blk, 0]
        w1 = win_ref[blk, 1]
        c0 = (w0 // CB) * CB
        nch = (w1 - c0 + CB - 1) // CB

        xblk = xblk_ref[...]
        rowb = brow_ref[...]
        sqr = sqr_ref[...]

        def merge_body(i, carry):
            bv, bi = carry
            ci = c0 // CB + i
            off = c0 + i * CB
            xw = xall_ref[pl.ds(off, CB), :]
            colidx = (off +
                      jax.lax.broadcasted_iota(jnp.int32, (1, CB), 1)
                      ).astype(jnp.float32)
            sqc = sqc_all_ref[pl.ds(ci, 1)].reshape(1, CB)
            dot = jax.lax.dot_general(
                xblk, xw, (((1,), (1,)), ((), ())),
                preferred_element_type=jnp.float32)
            dist = (sqr + sqc) - 2.0 * dot
            colb = bcol_ref[:, pl.ds(off, CB)]
            dist = jnp.where(rowb == colb, dist, MASKED)
            ich = jnp.broadcast_to(colidx, (RB, CB))
            return _knn_merge(bv, bi, dist, ich)

        bv0 = jnp.full((RB, BW), BIG, jnp.float32)
        bi0 = jnp.full((RB, BW), BIG, jnp.float32)
        _, bi = jax.lax.fori_loop(0, nch, merge_body, (bv0, bi0))

        biT = jnp.transpose(bi)

        def gather_body(i, accT):
            off = c0 + i * CB
            xtw = xt_ref[:, pl.ds(off, CB)]
            hi, lo, lo2 = _limbs(xtw)
            limbs = jnp.concatenate([hi, lo, lo2], axis=0)
            colc = (off +
                    jax.lax.broadcasted_iota(jnp.int32, (CB, 1), 0)
                    ).astype(jnp.float32)
            parts = []
            for t in range(KNN):
                ohT = (colc == biT[t:t + 1, :]).astype(jnp.bfloat16)
                g = jax.lax.dot_general(
                    limbs, ohT, (((1,), (0,)), ((), ())),
                    preferred_element_type=jnp.float32)
                parts.append(g[0:d_in] + g[d_in:2 * d_in]
                             + g[2 * d_in:3 * d_in])
            return accT + jnp.concatenate(parts, axis=0)

        xjT = jax.lax.fori_loop(
            0, nch, gather_body, jnp.zeros((KNN * d_in, RB), jnp.float32))
        pad_rows = (-(KNN * d_in)) % 8
        if pad_rows:
            xjT = jnp.concatenate(
                [xjT, jnp.zeros((pad_rows, RB), jnp.float32)], axis=0)
        xj = jnp.transpose(xjT)

        res = jnp.zeros((RB, d_out), jnp.float32)
        if split_mlp and n_layers == 1:
            w = wrefs[0][...]
            base = _dot(xblk, w[0:d_in] - w[d_in:2 * d_in]) + wrefs[1][...]
            for t in range(KNN):
                xjt = xj[:, t * d_in:(t + 1) * d_in]
                h = jnp.maximum(_dot(xjt, w[d_in:2 * d_in]) + base, 0.0)
                res = jnp.maximum(res, h)
        else:
            for t in range(KNN):
                xjt = xj[:, t * d_in:(t + 1) * d_in]
                h = jnp.concatenate([xblk, xjt - xblk], axis=1)
                for li in range(n_layers):
                    h = jnp.maximum(
                        _dot(h, wrefs[2 * li][...]) + wrefs[2 * li + 1][...],
                        0.0)
                res = jnp.maximum(res, h)
        out_ref[...] = res
    return body


def _edgeconv(x_pad, windows, brow, bcol, weights, d_in, d_out,
              split_mlp=False):
    n_layers = len(weights) // 2
    nblk = NP // RB
    nch_all = NP // CB
    sq = jnp.sum(x_pad * x_pad, axis=1)
    sq_row = sq.reshape(NP, 1)
    sq_col = sq.reshape(nch_all, 1, CB)
    in_specs = [
        pl.BlockSpec(memory_space=pltpu.SMEM),
        pl.BlockSpec((NP, d_in), lambda b: (0, 0)),
        pl.BlockSpec((d_in, NP), lambda b: (0, 0)),
        pl.BlockSpec((nch_all, 1, CB), lambda b: (0, 0, 0)),
        pl.BlockSpec((1, NP), lambda b: (0, 0)),
        pl.BlockSpec((RB, d_in), lambda b: (b, 0)),
        pl.BlockSpec((RB, 1), lambda b: (b, 0)),
        pl.BlockSpec((RB, 1), lambda b: (b, 0)),
    ] + [pl.BlockSpec(w.shape, lambda b: (0, 0)) for w in weights]
    return pl.pallas_call(
        _make_edgeconv_body(d_in, d_out, n_layers, split_mlp=split_mlp),
        grid=(nblk,),
        in_specs=in_specs,
        out_specs=pl.BlockSpec((RB, d_out), lambda b: (b, 0)),
        out_shape=jax.ShapeDtypeStruct((NP, d_out), jnp.float32),
    )(windows, x_pad, x_pad.T, sq_col, bcol, x_pad, sq_row, brow, *weights)


def _final_body(x1_ref, x2_ref, brow_ref, wa_ref, ba_ref,
                wh1_ref, bh1_ref, wh2_ref, bh2_ref, out_ref, pool_ref):
    blk = pl.program_id(0)
    nblk = pl.num_programs(0)

    @pl.when(blk == 0)
    def _():
        pool_ref[...] = jnp.full(pool_ref.shape, -jnp.inf, jnp.float32)

    g = jnp.concatenate([x1_ref[...], x2_ref[...]], axis=1)
    go = _dot(g, wa_ref[...]) + ba_ref[...]
    rowb = brow_ref[...]
    rows = []
    for gg in range(NSEG):
        m = jnp.where(rowb == jnp.float32(gg), go, -jnp.inf)
        rows.append(jnp.max(m, axis=0, keepdims=True))
    pool_ref[...] = jnp.maximum(pool_ref[...], jnp.concatenate(rows, axis=0))

    @pl.when(blk == nblk - 1)
    def _():
        pooled = pool_ref[...]
        h = jnp.maximum(_dot(pooled, wh1_ref[...]) + bh1_ref[...], 0.0)
        out_ref[...] = _dot(h, wh2_ref[...]) + bh2_ref[...]


def _finalize(x1, x2, brow, Wa, ba, Wh1, bh1, Wh2, bh2):
    nblk = NP // RB
    in_specs = [
        pl.BlockSpec((RB, x1.shape[1]), lambda b: (b, 0)),
        pl.BlockSpec((RB, x2.shape[1]), lambda b: (b, 0)),
        pl.BlockSpec((RB, 1), lambda b: (b, 0)),
    ] + [pl.BlockSpec(w.shape, lambda b: (0, 0))
         for w in (Wa, ba, Wh1, bh1, Wh2, bh2)]
    return pl.pallas_call(
        _final_body,
        grid=(nblk,),
        in_specs=in_specs,
        out_specs=pl.BlockSpec((NSEG, Wh2.shape[1]), lambda b: (0, 0)),
        out_shape=jax.ShapeDtypeStruct((NSEG, Wh2.shape[1]), jnp.float32),
        scratch_shapes=[pltpu.VMEM((NSEG, Wa.shape[1]), jnp.float32)],
    )(x1, x2, brow, Wa, ba, Wh1, bh1, Wh2, bh2)


def kernel(x, batch, W1a, b1a, W1b, b1b, W2a, b2a, Wa, ba, Wh1, bh1, Wh2, bh2):
    n = x.shape[0]
    pad = NP - n
    x_p = jnp.pad(x, ((0, pad), (0, 0)))
    batch_p = jnp.concatenate(
        [batch.astype(jnp.int32), jnp.full((pad,), NSEG, jnp.int32)])
    starts = jnp.searchsorted(
        batch_p, jnp.arange(NSEG + 2, dtype=jnp.int32)).astype(jnp.int32)
    rb_first = batch_p[::RB]
    rb_last = batch_p[RB - 1::RB]
    windows = jnp.stack(
        [starts[rb_first], starts[rb_last + 1]], axis=1).astype(jnp.int32)
    batch_f = batch_p.astype(jnp.float32)
    brow = batch_f.reshape(NP, 1)
    bcol = batch_f.reshape(1, NP)

    w1 = [W1a, b1a.reshape(1, -1), W1b, b1b.reshape(1, -1)]
    x1 = _edgeconv(x_p, windows, brow, bcol, w1, 3, 64)
    w2 = [W2a, b2a.reshape(1, -1)]
    x2 = _edgeconv(x1, windows, brow, bcol, w2, 64, 128)
    out = _finalize(x1, x2, brow,
                    Wa, ba.reshape(1, -1), Wh1, bh1.reshape(1, -1),
                    Wh2, bh2.reshape(1, -1))
    return out


# speedup vs baseline: 1.4028x; 1.4028x over previous
import jax
import jax.numpy as jnp
from jax.experimental import pallas as pl
from jax.experimental.pallas import tpu as pltpu

NP = 10240
RB = 512
CB = 512
KNN = 20
BW = 32
NSEG = 16
BIG = float(3e38)
MASKED = float(1e30)


def _knn_merge(bv, bi, dch, ich):
    cv = jnp.concatenate([bv, dch], axis=1)
    ci = jnp.concatenate([bi, ich], axis=1)
    ms, cs = [], []
    for _ in range(KNN):
        m = jnp.min(cv, axis=1, keepdims=True)
        tie = jnp.where(cv == m, ci, BIG)
        c = jnp.min(tie, axis=1, keepdims=True)
        sel = ci == c
        cv = jnp.where(sel, BIG, cv)
        ms.append(m)
        cs.append(c)
    rb = bv.shape[0]
    ms.append(jnp.full((rb, BW - KNN), BIG, jnp.float32))
    cs.append(jnp.full((rb, BW - KNN), BIG, jnp.float32))
    return jnp.concatenate(ms, axis=1), jnp.concatenate(cs, axis=1)


def _dot(a, b):
    return jax.lax.dot_general(a, b, (((1,), (0,)), ((), ())),
                               preferred_element_type=jnp.float32)


def _limbs(x):
    hi = x.astype(jnp.bfloat16)
    r1 = x - hi.astype(jnp.float32)
    lo = r1.astype(jnp.bfloat16)
    r2 = r1 - lo.astype(jnp.float32)
    lo2 = r2.astype(jnp.bfloat16)
    return hi, lo, lo2


def _make_edgeconv_body(d_in, d_out, n_layers, split_mlp=False):
    def body(win_ref, xall_ref, xt_ref, sqc_all_ref, bcol_ref,
             xblk_ref, sqr_ref, brow_ref, *rest):
        wrefs = rest[:-1]
        out_ref = rest[-1]
        blk = pl.program_id(0)
        w0 = win_ref[blk, 0]
        w1 = win_ref[blk, 1]
        c0 = (w0 // CB) * CB
        nch = (w1 - c0 + CB - 1) // CB

        xblk = xblk_ref[...]
        rowb = brow_ref[...]
        sqr = sqr_ref[...]

        def merge_body(i, carry):
            bv, bi = carry
            ci = c0 // CB + i
            off = c0 + i * CB
            xw = xall_ref[pl.ds(off, CB), :]
            colidx = (off +
                      jax.lax.broadcasted_iota(jnp.int32, (1, CB), 1)
                      ).astype(jnp.float32)
            sqc = sqc_all_ref[pl.ds(ci, 1)].reshape(1, CB)
            dot = jax.lax.dot_general(
                xblk, xw, (((1,), (1,)), ((), ())),
                preferred_element_type=jnp.float32)
            dist = (sqr + sqc) - 2.0 * dot
            colb = bcol_ref[:, pl.ds(off, CB)]
            dist = jnp.where(rowb == colb, dist, MASKED)
            ich = jnp.broadcast_to(colidx, (RB, CB))
            return _knn_merge(bv, bi, dist, ich)

        bv0 = jnp.full((RB, BW), BIG, jnp.float32)
        bi0 = jnp.full((RB, BW), BIG, jnp.float32)
        _, bi = jax.lax.fori_loop(0, nch, merge_body, (bv0, bi0))

        biT = jnp.transpose(bi)

        def gather_body(i, accT):
            off = c0 + i * CB
            xtw = xt_ref[:, pl.ds(off, CB)]
            hi, lo, lo2 = _limbs(xtw)
            limbs = jnp.concatenate([hi, lo, lo2], axis=0)
            colc = (off +
                    jax.lax.broadcasted_iota(jnp.int32, (CB, 1), 0)
                    ).astype(jnp.float32)
            parts = []
            for t in range(KNN):
                ohT = (colc == biT[t:t + 1, :]).astype(jnp.bfloat16)
                g = jax.lax.dot_general(
                    limbs, ohT, (((1,), (0,)), ((), ())),
                    preferred_element_type=jnp.float32)
                parts.append(g[0:d_in] + g[d_in:2 * d_in]
                             + g[2 * d_in:3 * d_in])
            return accT + jnp.concatenate(parts, axis=0)

        xjT = jax.lax.fori_loop(
            0, nch, gather_body, jnp.zeros((KNN * d_in, RB), jnp.float32))
        pad_rows = (-(KNN * d_in)) % 8
        if pad_rows:
            xjT = jnp.concatenate(
                [xjT, jnp.zeros((pad_rows, RB), jnp.float32)], axis=0)
        xj = jnp.transpose(xjT)

        res = jnp.zeros((RB, d_out), jnp.float32)
        if split_mlp and n_layers == 1:
            w = wrefs[0][...]
            base = _dot(xblk, w[0:d_in] - w[d_in:2 * d_in]) + wrefs[1][...]
            for t in range(KNN):
                xjt = xj[:, t * d_in:(t + 1) * d_in]
                h = jnp.maximum(_dot(xjt, w[d_in:2 * d_in]) + base, 0.0)
                res = jnp.maximum(res, h)
        else:
            for t in range(KNN):
                xjt = xj[:, t * d_in:(t + 1) * d_in]
                h = jnp.concatenate([xblk, xjt - xblk], axis=1)
                for li in range(n_layers):
                    h = jnp.maximum(
                        _dot(h, wrefs[2 * li][...]) + wrefs[2 * li + 1][...],
                        0.0)
                res = jnp.maximum(res, h)
        out_ref[...] = res
    return body


def _edgeconv(x_pad, windows, brow, bcol, weights, d_in, d_out,
              split_mlp=False):
    n_layers = len(weights) // 2
    nblk = NP // RB
    nch_all = NP // CB
    sq = jnp.sum(x_pad * x_pad, axis=1)
    sq_row = sq.reshape(NP, 1)
    sq_col = sq.reshape(nch_all, 1, CB)
    in_specs = [
        pl.BlockSpec(memory_space=pltpu.SMEM),
        pl.BlockSpec((NP, d_in), lambda b: (0, 0)),
        pl.BlockSpec((d_in, NP), lambda b: (0, 0)),
        pl.BlockSpec((nch_all, 1, CB), lambda b: (0, 0, 0)),
        pl.BlockSpec((1, NP), lambda b: (0, 0)),
        pl.BlockSpec((RB, d_in), lambda b: (b, 0)),
        pl.BlockSpec((RB, 1), lambda b: (b, 0)),
        pl.BlockSpec((RB, 1), lambda b: (b, 0)),
    ] + [pl.BlockSpec(w.shape, lambda b: (0, 0)) for w in weights]
    return pl.pallas_call(
        _make_edgeconv_body(d_in, d_out, n_layers, split_mlp=split_mlp),
        grid=(nblk,),
        in_specs=in_specs,
        out_specs=pl.BlockSpec((RB, d_out), lambda b: (b, 0)),
        out_shape=jax.ShapeDtypeStruct((NP, d_out), jnp.float32),
    )(windows, x_pad, x_pad.T, sq_col, bcol, x_pad, sq_row, brow, *weights)


def _final_body(x1_ref, x2_ref, brow_ref, wa_ref, ba_ref,
                wh1_ref, bh1_ref, wh2_ref, bh2_ref, out_ref, pool_ref):
    blk = pl.program_id(0)
    nblk = pl.num_programs(0)

    @pl.when(blk == 0)
    def _():
        pool_ref[...] = jnp.full(pool_ref.shape, -jnp.inf, jnp.float32)

    g = jnp.concatenate([x1_ref[...], x2_ref[...]], axis=1)
    go = _dot(g, wa_ref[...]) + ba_ref[...]
    rowb = brow_ref[...]
    rows = []
    for gg in range(NSEG):
        m = jnp.where(rowb == jnp.float32(gg), go, -jnp.inf)
        rows.append(jnp.max(m, axis=0, keepdims=True))
    pool_ref[...] = jnp.maximum(pool_ref[...], jnp.concatenate(rows, axis=0))

    @pl.when(blk == nblk - 1)
    def _():
        pooled = pool_ref[...]
        h = jnp.maximum(_dot(pooled, wh1_ref[...]) + bh1_ref[...], 0.0)
        out_ref[...] = _dot(h, wh2_ref[...]) + bh2_ref[...]


def _finalize(x1, x2, brow, Wa, ba, Wh1, bh1, Wh2, bh2):
    nblk = NP // RB
    in_specs = [
        pl.BlockSpec((RB, x1.shape[1]), lambda b: (b, 0)),
        pl.BlockSpec((RB, x2.shape[1]), lambda b: (b, 0)),
        pl.BlockSpec((RB, 1), lambda b: (b, 0)),
    ] + [pl.BlockSpec(w.shape, lambda b: (0, 0))
         for w in (Wa, ba, Wh1, bh1, Wh2, bh2)]
    return pl.pallas_call(
        _final_body,
        grid=(nblk,),
        in_specs=in_specs,
        out_specs=pl.BlockSpec((NSEG, Wh2.shape[1]), lambda b: (0, 0)),
        out_shape=jax.ShapeDtypeStruct((NSEG, Wh2.shape[1]), jnp.float32),
        scratch_shapes=[pltpu.VMEM((NSEG, Wa.shape[1]), jnp.float32)],
    )(x1, x2, brow, Wa, ba, Wh1, bh1, Wh2, bh2)


def kernel(x, batch, W1a, b1a, W1b, b1b, W2a, b2a, Wa, ba, Wh1, bh1, Wh2, bh2):
    n = x.shape[0]
    pad = NP - n
    x_p = jnp.pad(x, ((0, pad), (0, 0)))
    batch_p = jnp.concatenate(
        [batch.astype(jnp.int32), jnp.full((pad,), NSEG, jnp.int32)])
    starts = jnp.searchsorted(
        batch_p, jnp.arange(NSEG + 2, dtype=jnp.int32)).astype(jnp.int32)
    rb_first = batch_p[::RB]
    rb_last = batch_p[RB - 1::RB]
    windows = jnp.stack(
        [starts[rb_first], starts[rb_last + 1]], axis=1).astype(jnp.int32)
    batch_f = batch_p.astype(jnp.float32)
    brow = batch_f.reshape(NP, 1)
    bcol = batch_f.reshape(1, NP)

    w1 = [W1a, b1a.reshape(1, -1), W1b, b1b.reshape(1, -1)]
    x1 = _edgeconv(x_p, windows, brow, bcol, w1, 3, 64)
    w2 = [W2a, b2a.reshape(1, -1)]
    x2 = _edgeconv(x1, windows, brow, bcol, w2, 64, 128)
    out = _finalize(x1, x2, brow,
                    Wa, ba.reshape(1, -1), Wh1, bh1.reshape(1, -1),
                    Wh2, bh2.reshape(1, -1))
    return out


# speedup vs baseline: 1.4030x; 1.0002x over previous
import jax
import jax.numpy as jnp
from jax.experimental import pallas as pl
from jax.experimental.pallas import tpu as pltpu

NP = 10240
RB = 512
CB = 512
KNN = 20
BW = 32
NSEG = 16
BIG = float(3e38)
MASKED = float(1e30)


def _knn_merge(bv, bi, dch, ich):
    cv = jnp.concatenate([bv, dch], axis=1)
    ci = jnp.concatenate([bi, ich], axis=1)
    ms, cs = [], []
    for _ in range(KNN):
        m = jnp.min(cv, axis=1, keepdims=True)
        tie = jnp.where(cv == m, ci, BIG)
        c = jnp.min(tie, axis=1, keepdims=True)
        sel = ci == c
        cv = jnp.where(sel, BIG, cv)
        ms.append(m)
        cs.append(c)
    rb = bv.shape[0]
    ms.append(jnp.full((rb, BW - KNN), BIG, jnp.float32))
    cs.append(jnp.full((rb, BW - KNN), BIG, jnp.float32))
    return jnp.concatenate(ms, axis=1), jnp.concatenate(cs, axis=1)


def _dot(a, b):
    return jax.lax.dot_general(a, b, (((1,), (0,)), ((), ())),
                               preferred_element_type=jnp.float32)


def _limbs(x):
    hi = x.astype(jnp.bfloat16)
    r1 = x - hi.astype(jnp.float32)
    lo = r1.astype(jnp.bfloat16)
    r2 = r1 - lo.astype(jnp.float32)
    lo2 = r2.astype(jnp.bfloat16)
    return hi, lo, lo2


def _make_edgeconv_body(d_in, d_out, n_layers):
    def body(win_ref, xall_ref, xt_ref, sqc_all_ref, bcol_ref,
             xblk_ref, sqr_ref, brow_ref, *rest):
        wrefs = rest[:-1]
        out_ref = rest[-1]
        blk = pl.program_id(0)
        w0 = win_ref[blk, 0]
        w1 = win_ref[blk, 1]
        c0 = (w0 // CB) * CB
        nch = (w1 - c0 + CB - 1) // CB

        xblk = xblk_ref[...]
        rowb = brow_ref[...]
        sqr = sqr_ref[...]

        def merge_body(i, carry):
            bv, bi = carry
            ci = c0 // CB + i
            off = c0 + i * CB
            xw = xall_ref[pl.ds(off, CB), :]
            colidx = (off +
                      jax.lax.broadcasted_iota(jnp.int32, (1, CB), 1)
                      ).astype(jnp.float32)
            sqc = sqc_all_ref[pl.ds(ci, 1)].reshape(1, CB)
            dot = jax.lax.dot_general(
                xblk, xw, (((1,), (1,)), ((), ())),
                preferred_element_type=jnp.float32)
            dist = (sqr + sqc) - 2.0 * dot
            colb = bcol_ref[:, pl.ds(off, CB)]
            dist = jnp.where(rowb == colb, dist, MASKED)
            ich = jnp.broadcast_to(colidx, (RB, CB))
            return _knn_merge(bv, bi, dist, ich)

        bv0 = jnp.full((RB, BW), BIG, jnp.float32)
        bi0 = jnp.full((RB, BW), BIG, jnp.float32)
        _, bi = jax.lax.fori_loop(0, nch, merge_body, (bv0, bi0))

        biT = jnp.transpose(bi)

        def gather_body(i, accT):
            off = c0 + i * CB
            xtw = xt_ref[:, pl.ds(off, CB)]
            hi, lo, lo2 = _limbs(xtw)
            limbs = jnp.concatenate([hi, lo, lo2], axis=0)
            colc = (off +
                    jax.lax.broadcasted_iota(jnp.int32, (CB, 1), 0)
                    ).astype(jnp.float32)
            parts = []
            for t in range(KNN):
                ohT = (colc == biT[t:t + 1, :]).astype(jnp.bfloat16)
                g = jax.lax.dot_general(
                    limbs, ohT, (((1,), (0,)), ((), ())),
                    preferred_element_type=jnp.float32)
                parts.append(g[0:d_in] + g[d_in:2 * d_in]
                             + g[2 * d_in:3 * d_in])
            return accT + jnp.concatenate(parts, axis=0)

        xjT = jax.lax.fori_loop(
            0, nch, gather_body, jnp.zeros((KNN * d_in, RB), jnp.float32))
        pad_rows = (-(KNN * d_in)) % 8
        if pad_rows:
            xjT = jnp.concatenate(
                [xjT, jnp.zeros((pad_rows, RB), jnp.float32)], axis=0)
        xj = jnp.transpose(xjT)

        res = jnp.zeros((RB, d_out), jnp.float32)
        for t in range(KNN):
            xjt = xj[:, t * d_in:(t + 1) * d_in]
            h = jnp.concatenate([xblk, xjt - xblk], axis=1)
            for li in range(n_layers):
                h = jnp.maximum(
                    _dot(h, wrefs[2 * li][...]) + wrefs[2 * li + 1][...], 0.0)
            res = jnp.maximum(res, h)
        out_ref[...] = res
    return body


def _edgeconv(x_pad, windows, brow, bcol, weights, d_in, d_out):
    n_layers = len(weights) // 2
    nblk = NP // RB
    nch_all = NP // CB
    sq = jnp.sum(x_pad * x_pad, axis=1)
    sq_row = sq.reshape(NP, 1)
    sq_col = sq.reshape(nch_all, 1, CB)
    in_specs = [
        pl.BlockSpec(memory_space=pltpu.SMEM),
        pl.BlockSpec((NP, d_in), lambda b: (0, 0)),
        pl.BlockSpec((d_in, NP), lambda b: (0, 0)),
        pl.BlockSpec((nch_all, 1, CB), lambda b: (0, 0, 0)),
        pl.BlockSpec((1, NP), lambda b: (0, 0)),
        pl.BlockSpec((RB, d_in), lambda b: (b, 0)),
        pl.BlockSpec((RB, 1), lambda b: (b, 0)),
        pl.BlockSpec((RB, 1), lambda b: (b, 0)),
    ] + [pl.BlockSpec(w.shape, lambda b: (0, 0)) for w in weights]
    return pl.pallas_call(
        _make_edgeconv_body(d_in, d_out, n_layers),
        grid=(nblk,),
        in_specs=in_specs,
        out_specs=pl.BlockSpec((RB, d_out), lambda b: (b, 0)),
        out_shape=jax.ShapeDtypeStruct((NP, d_out), jnp.float32),
    )(windows, x_pad, x_pad.T, sq_col, bcol, x_pad, sq_row, brow, *weights)


def _final_body(x1_ref, x2_ref, brow_ref, wa_ref, ba_ref,
                wh1_ref, bh1_ref, wh2_ref, bh2_ref, out_ref, pool_ref):
    blk = pl.program_id(0)
    nblk = pl.num_programs(0)

    @pl.when(blk == 0)
    def _():
        pool_ref[...] = jnp.full(pool_ref.shape, -jnp.inf, jnp.float32)

    g = jnp.concatenate([x1_ref[...], x2_ref[...]], axis=1)
    go = _dot(g, wa_ref[...]) + ba_ref[...]
    rowb = brow_ref[...]
    rows = []
    for gg in range(NSEG):
        m = jnp.where(rowb == jnp.float32(gg), go, -jnp.inf)
        rows.append(jnp.max(m, axis=0, keepdims=True))
    pool_ref[...] = jnp.maximum(pool_ref[...], jnp.concatenate(rows, axis=0))

    @pl.when(blk == nblk - 1)
    def _():
        pooled = pool_ref[...]
        h = jnp.maximum(_dot(pooled, wh1_ref[...]) + bh1_ref[...], 0.0)
        out_ref[...] = _dot(h, wh2_ref[...]) + bh2_ref[...]


def _finalize(x1, x2, brow, Wa, ba, Wh1, bh1, Wh2, bh2):
    nblk = NP // RB
    in_specs = [
        pl.BlockSpec((RB, x1.shape[1]), lambda b: (b, 0)),
        pl.BlockSpec((RB, x2.shape[1]), lambda b: (b, 0)),
        pl.BlockSpec((RB, 1), lambda b: (b, 0)),
    ] + [pl.BlockSpec(w.shape, lambda b: (0, 0))
         for w in (Wa, ba, Wh1, bh1, Wh2, bh2)]
    return pl.pallas_call(
        _final_body,
        grid=(nblk,),
        in_specs=in_specs,
        out_specs=pl.BlockSpec((NSEG, Wh2.shape[1]), lambda b: (0, 0)),
        out_shape=jax.ShapeDtypeStruct((NSEG, Wh2.shape[1]), jnp.float32),
        scratch_shapes=[pltpu.VMEM((NSEG, Wa.shape[1]), jnp.float32)],
    )(x1, x2, brow, Wa, ba, Wh1, bh1, Wh2, bh2)


def kernel(x, batch, W1a, b1a, W1b, b1b, W2a, b2a, Wa, ba, Wh1, bh1, Wh2, bh2):
    n = x.shape[0]
    pad = NP - n
    x_p = jnp.pad(x, ((0, pad), (0, 0)))
    batch_p = jnp.concatenate(
        [batch.astype(jnp.int32), jnp.full((pad,), NSEG, jnp.int32)])
    starts = jnp.searchsorted(
        batch_p, jnp.arange(NSEG + 2, dtype=jnp.int32)).astype(jnp.int32)
    rb_first = batch_p[::RB]
    rb_last = batch_p[RB - 1::RB]
    windows = jnp.stack(
        [starts[rb_first], starts[rb_last + 1]], axis=1).astype(jnp.int32)
    batch_f = batch_p.astype(jnp.float32)
    brow = batch_f.reshape(NP, 1)
    bcol = batch_f.reshape(1, NP)

    w1 = [W1a, b1a.reshape(1, -1), W1b, b1b.reshape(1, -1)]
    x1 = _edgeconv(x_p, windows, brow, bcol, w1, 3, 64)
    w2 = [W2a, b2a.reshape(1, -1)]
    x2 = _edgeconv(x1, windows, brow, bcol, w2, 64, 128)
    out = _finalize(x1, x2, brow,
                    Wa, ba.reshape(1, -1), Wh1, bh1.reshape(1, -1),
                    Wh2, bh2.reshape(1, -1))
    return out
